# baseline (speedup 1.0000x reference)
import functools

import jax
import jax.numpy as jnp
from jax import lax
from jax.experimental import pallas as pl
from jax.experimental.pallas import tpu as pltpu
from jax.experimental.pallas import tpu_sc as plsc

_NC = 2
_NS = 16
_NW = _NC * _NS
_BB = 128
_L = 16


@functools.partial(jax.jit, static_argnames=("hist", "nblk", "dim"))
def _gather_call(idx, table, hist, nblk, dim):
    dt = dim // 8
    n_chunks = hist * nblk
    per_w = n_chunks // _NW
    mesh = plsc.VectorSubcoreMesh(core_axis_name="c", subcore_axis_name="s")

    @functools.partial(
        pl.kernel,
        mesh=mesh,
        compiler_params=pltpu.CompilerParams(use_tc_tiling_on_sc=False,
                                             needs_layout_passes=False),
        out_type=jax.ShapeDtypeStruct((hist, dt, nblk, 8, _BB), jnp.float32),
        scratch_types=[
            pltpu.VMEM((_BB,), jnp.int32),
            pltpu.VMEM((_BB,), jnp.int32),
            pltpu.VMEM((_BB, dim), jnp.float32),
            pltpu.VMEM((_BB, dim), jnp.float32),
            pltpu.VMEM((dim, _BB), jnp.float32),
            pltpu.VMEM((dim, _BB), jnp.float32),
            pltpu.SemaphoreType.DMA,
            pltpu.SemaphoreType.DMA,
            pltpu.SemaphoreType.DMA,
            pltpu.SemaphoreType.DMA,
        ],
    )
    def emb(idx_hbm, tbl_hbm, out_hbm,
            idx0, idx1, rows0, rows1, rt0, rt1, gsem0, gsem1, wsem0, wsem1):
        wid = lax.axis_index("s") * _NC + lax.axis_index("c")
        cid0 = wid * per_w
        idxs, rows = (idx0, idx1), (rows0, rows1)
        rts = (rt0, rt1)
        gsems, wsems = (gsem0, gsem1), (wsem0, wsem1)
        cvecs = [lax.iota(jnp.int32, _L) + cb * _L for cb in range(_BB // _L)]

        def fire_gather(g, b):
            cid = cid0 + g
            pltpu.sync_copy(idx_hbm.at[cid], idxs[b])
            pltpu.async_copy(tbl_hbm.at[idxs[b]], rows[b], gsems[b])

        def drain_gather(b):
            pltpu.make_async_copy(tbl_hbm.at[pl.ds(0, _BB)], rows[b],
                                  gsems[b]).wait()

        def transpose(b):
            rv, rt = rows[b], rts[b]

            def do_body(do, carry):
                for r in range(8):
                    d = do * 8 + r
                    dvec = jnp.zeros((_L,), jnp.int32) + d
                    for cb in range(_BB // _L):
                        v = plsc.load_gather(rv, [cvecs[cb], dvec])
                        plsc.store_scatter(rt, [dvec, cvecs[cb]], v)
                return carry

            lax.fori_loop(0, dt, do_body, 0)

        def fire_writes(g, b):
            cid = cid0 + g
            h = cid // nblk
            tc = cid - h * nblk
            for do in range(dt):
                pltpu.async_copy(rts[b].at[pl.ds(do * 8, 8)],
                                 out_hbm.at[h, do, tc], wsems[b])

        def drain_writes(b):
            for do in range(dt):
                pltpu.make_async_copy(rts[b].at[pl.ds(0, 8)],
                                      out_hbm.at[0, 0, 0], wsems[b]).wait()

        fire_gather(0, 0)

        def body(h_it, carry):
            first = h_it < 1
            g0 = 2 * h_it
            fire_gather(g0 + 1, 1)
            drain_gather(0)
            pl.when(jnp.logical_not(first))(lambda: drain_writes(0))
            transpose(0)
            fire_writes(g0, 0)
            pl.when(g0 + 2 < per_w)(lambda: fire_gather(g0 + 2, 0))
            drain_gather(1)
            pl.when(jnp.logical_not(first))(lambda: drain_writes(1))
            transpose(1)
            fire_writes(g0 + 1, 1)
            return carry

        lax.fori_loop(0, per_w // 2, body, 0)
        drain_writes(0)
        drain_writes(1)

    return emb(idx, table)


def kernel(input_ids, table):
    batch, hist = input_ids.shape
    vocab, dim = table.shape
    nblk = batch // _BB

    idx = input_ids.T.reshape(hist * nblk, _BB).astype(jnp.int32)
    out5 = _gather_call(idx, table, hist, nblk, dim)
    return out5.transpose(2, 4, 0, 1, 3).reshape(batch, hist, dim)


# speedup vs baseline: 1.1956x; 1.1956x over previous
import functools

import jax
import jax.numpy as jnp
from jax import lax
from jax.experimental import pallas as pl
from jax.experimental.pallas import tpu as pltpu
from jax.experimental.pallas import tpu_sc as plsc

_NC = 2
_NS = 16
_NW = _NC * _NS
_BB = 128
_L = 16


@functools.partial(jax.jit, static_argnames=("hist", "nblk", "dim"))
def _gather_call(idx, table, hist, nblk, dim):
    dt = dim // 8
    tile = 8 * _BB
    n_chunks = hist * nblk
    per_w = n_chunks // _NW
    mesh = plsc.VectorSubcoreMesh(core_axis_name="c", subcore_axis_name="s")

    @functools.partial(
        pl.kernel,
        mesh=mesh,
        compiler_params=pltpu.CompilerParams(use_tc_tiling_on_sc=False,
                                             needs_layout_passes=False),
        out_type=jax.ShapeDtypeStruct((hist, dt, nblk, tile), jnp.float32),
        scratch_types=[
            pltpu.VMEM((_BB,), jnp.int32),
            pltpu.VMEM((_BB,), jnp.int32),
            pltpu.VMEM((_BB, dim), jnp.float32),
            pltpu.VMEM((_BB, dim), jnp.float32),
            pltpu.VMEM((dim * _BB,), jnp.float32),
            pltpu.VMEM((dim * _BB,), jnp.float32),
            pltpu.SemaphoreType.DMA,
            pltpu.SemaphoreType.DMA,
            pltpu.SemaphoreType.DMA,
            pltpu.SemaphoreType.DMA,
        ],
    )
    def emb(idx_hbm, tbl_hbm, out_hbm,
            idx0, idx1, rows0, rows1, rt0, rt1, gsem0, gsem1, wsem0, wsem1):
        wid = lax.axis_index("s") * _NC + lax.axis_index("c")
        cid0 = wid * per_w
        idxs, rows = (idx0, idx1), (rows0, rows1)
        rts = (rt0, rt1)
        gsems, wsems = (gsem0, gsem1), (wsem0, wsem1)
        ivecs = [(lax.iota(jnp.int32, _L) + db * _L) * _BB
                 for db in range(dim // _L)]

        def fire_gather(g, b):
            cid = cid0 + g
            pltpu.sync_copy(idx_hbm.at[cid], idxs[b])
            pltpu.async_copy(tbl_hbm.at[idxs[b]], rows[b], gsems[b])

        def drain_gather(b):
            pltpu.make_async_copy(tbl_hbm.at[pl.ds(0, _BB)], rows[b],
                                  gsems[b]).wait()

        def transpose(b):
            rv, rt = rows[b], rts[b]
            for c in range(_BB):
                for db in range(dim // _L):
                    v = rv[c, pl.ds(db * _L, _L)]
                    plsc.store_scatter(rt, [ivecs[db] + c], v)

        def fire_writes(g, b):
            cid = cid0 + g
            h = cid // nblk
            tc = cid - h * nblk
            for do in range(dt):
                pltpu.async_copy(rts[b].at[pl.ds(do * tile, tile)],
                                 out_hbm.at[h, do, tc], wsems[b])

        def drain_writes(b):
            for do in range(dt):
                pltpu.make_async_copy(rts[b].at[pl.ds(0, tile)],
                                      out_hbm.at[0, 0, 0], wsems[b]).wait()

        fire_gather(0, 0)

        def body(h_it, carry):
            first = h_it < 1
            g0 = 2 * h_it
            fire_gather(g0 + 1, 1)
            drain_gather(0)
            pl.when(jnp.logical_not(first))(lambda: drain_writes(0))
            transpose(0)
            fire_writes(g0, 0)
            pl.when(g0 + 2 < per_w)(lambda: fire_gather(g0 + 2, 0))
            drain_gather(1)
            pl.when(jnp.logical_not(first))(lambda: drain_writes(1))
            transpose(1)
            fire_writes(g0 + 1, 1)
            return carry

        lax.fori_loop(0, per_w // 2, body, 0)
        drain_writes(0)
        drain_writes(1)

    return emb(idx, table)


def kernel(input_ids, table):
    batch, hist = input_ids.shape
    vocab, dim = table.shape
    nblk = batch // _BB

    idx = input_ids.T.reshape(hist * nblk, _BB).astype(jnp.int32)
    out4 = _gather_call(idx, table, hist, nblk, dim)
    out5 = out4.reshape(hist, dim // 8, nblk, 8, _BB)
    return out5.transpose(2, 4, 0, 1, 3).reshape(batch, hist, dim)


# speedup vs baseline: 1.3785x; 1.1529x over previous
import functools

import jax
import jax.numpy as jnp
from jax import lax
from jax.experimental import pallas as pl
from jax.experimental.pallas import tpu as pltpu
from jax.experimental.pallas import tpu_sc as plsc

_NC = 2
_NS = 16
_NW = _NC * _NS
_BB = 128
_L = 16
_NR = 8
_GD = 4


@functools.partial(jax.jit, static_argnames=("hist", "nblk", "dim"))
def _gather_call(idx, table, hist, nblk, dim):
    dt = dim // 8
    tile = 8 * _BB
    n_chunks = hist * nblk
    per_w = n_chunks // _NW
    assert per_w % _NR == 0
    mesh = plsc.VectorSubcoreMesh(core_axis_name="c", subcore_axis_name="s")

    @functools.partial(
        pl.kernel,
        mesh=mesh,
        compiler_params=pltpu.CompilerParams(use_tc_tiling_on_sc=False,
                                             needs_layout_passes=False),
        out_type=jax.ShapeDtypeStruct((hist, dt, nblk, tile), jnp.float32),
        scratch_types=[
            pltpu.VMEM((_NR, _BB), jnp.int32),
            pltpu.VMEM((_NR, _BB, dim), jnp.float32),
            pltpu.VMEM((2, dim * _BB), jnp.float32),
            [pltpu.SemaphoreType.DMA] * _NR,
            [pltpu.SemaphoreType.DMA] * _NR,
            [pltpu.SemaphoreType.DMA] * 2,
        ],
    )
    def emb(idx_hbm, tbl_hbm, out_hbm, idx_v, rows_v, rt_v,
            isems, gsems, wsems):
        wid = lax.axis_index("s") * _NC + lax.axis_index("c")
        cid0 = wid * per_w
        ivecs = [(lax.iota(jnp.int32, _L) + db * _L) * _BB
                 for db in range(dim // _L)]

        def fire_idx(g, s):
            pltpu.async_copy(idx_hbm.at[cid0 + g], idx_v.at[s], isems[s])

        def wait_idx(s):
            pltpu.make_async_copy(idx_hbm.at[0], idx_v.at[s], isems[s]).wait()

        def fire_gather(s):
            pltpu.async_copy(tbl_hbm.at[idx_v.at[s]], rows_v.at[s], gsems[s])

        def drain_gather(s):
            pltpu.make_async_copy(tbl_hbm.at[pl.ds(0, _BB)], rows_v.at[s],
                                  gsems[s]).wait()

        def transpose(s, b):
            rv = rows_v.at[s]
            rt = rt_v.at[b]

            def col_body(c0, carry):
                c_base = c0 * 8
                vs = []
                for cc in range(8):
                    c = c_base + cc
                    for db in range(dim // _L):
                        vs.append((c, db, rv[c, pl.ds(db * _L, _L)]))
                for c, db, v in vs:
                    plsc.store_scatter(rt, [ivecs[db] + c], v)
                return carry

            lax.fori_loop(0, _BB // 8, col_body, 0)

        def fire_writes(g, b):
            cid = cid0 + g
            h = cid // nblk
            tc = cid - h * nblk
            for do in range(dt):
                pltpu.async_copy(rt_v.at[b, pl.ds(do * tile, tile)],
                                 out_hbm.at[h, do, tc], wsems[b])

        def drain_writes(b):
            pltpu.make_async_copy(tbl_hbm.at[pl.ds(0, _BB)], rows_v.at[0],
                                  wsems[b]).wait()

        for s in range(_NR):
            fire_idx(s, s)
        for s in range(_GD):
            wait_idx(s)
            fire_gather(s)

        def body(h_it, carry):
            g_base = _NR * h_it
            for s in range(_NR):
                g = g_base + s
                b = s % 2
                drain_gather(s)
                pl.when((h_it > 0) | (s >= 2))(lambda b=b: drain_writes(b))
                transpose(s, b)
                fire_writes(g, b)
                pl.when(g + _NR < per_w)(lambda g=g, s=s: fire_idx(g + _NR, s))
                u = (s + _GD) % _NR

                def launch(u=u):
                    wait_idx(u)
                    fire_gather(u)

                pl.when(g + _GD < per_w)(launch)
            return carry

        lax.fori_loop(0, per_w // _NR, body, 0)
        drain_writes(0)
        drain_writes(1)

    return emb(idx, table)


def kernel(input_ids, table):
    batch, hist = input_ids.shape
    vocab, dim = table.shape
    nblk = batch // _BB

    idx = input_ids.T.reshape(hist * nblk, _BB).astype(jnp.int32)
    out4 = _gather_call(idx, table, hist, nblk, dim)
    out5 = out4.reshape(hist, dim // 8, nblk, 8, _BB)
    return out5.transpose(2, 4, 0, 1, 3).reshape(batch, hist, dim)


# speedup vs baseline: 3.5579x; 2.5811x over previous
import functools

import jax
import jax.numpy as jnp
from jax import lax
from jax.experimental import pallas as pl
from jax.experimental.pallas import tpu as pltpu
from jax.experimental.pallas import tpu_sc as plsc

_NC = 2
_NS = 16
_NW = _NC * _NS
_BB = 128
_L = 16
_NR = 8
_GD = 4


@functools.partial(jax.jit, static_argnames=("hist", "nblk", "dim"))
def _gather_call(idx, table, hist, nblk, dim):
    dt = dim // 8
    tile = 8 * _BB
    n_chunks = hist * nblk
    per_w = n_chunks // _NW
    assert per_w % _NR == 0
    mesh = plsc.VectorSubcoreMesh(core_axis_name="c", subcore_axis_name="s")

    @functools.partial(
        pl.kernel,
        mesh=mesh,
        compiler_params=pltpu.CompilerParams(use_tc_tiling_on_sc=False,
                                             needs_layout_passes=False),
        out_type=jax.ShapeDtypeStruct((hist, dt, nblk, 8, _BB), jnp.float32),
        scratch_types=[
            pltpu.VMEM((_NR, _BB), jnp.int32),
            pltpu.VMEM((_NR, _BB, dim), jnp.float32),
            pltpu.VMEM((2, dim, _BB + 1), jnp.float32),
            [pltpu.SemaphoreType.DMA] * _NR,
            [pltpu.SemaphoreType.DMA] * _NR,
            [pltpu.SemaphoreType.DMA] * 2,
        ],
    )
    def emb(idx_hbm, tbl_hbm, out_hbm, idx_v, rows_v, rt_v,
            isems, gsems, wsems):
        wid = lax.axis_index("s") * _NC + lax.axis_index("c")
        cid0 = wid * per_w
        dvecs = [lax.iota(jnp.int32, _L) + db * _L for db in range(dim // _L)]

        def fire_idx(g, s):
            pltpu.async_copy(idx_hbm.at[cid0 + g], idx_v.at[s], isems[s])

        def wait_idx(s):
            pltpu.make_async_copy(idx_hbm.at[0], idx_v.at[s], isems[s]).wait()

        def fire_gather(s):
            pltpu.async_copy(tbl_hbm.at[idx_v.at[s]], rows_v.at[s], gsems[s])

        def drain_gather(s):
            pltpu.make_async_copy(tbl_hbm.at[pl.ds(0, _BB)], rows_v.at[s],
                                  gsems[s]).wait()

        def transpose(s, b):
            rv = rows_v.at[s]
            rt = rt_v.at[b]

            def col_body(c0, carry):
                c_base = c0 * 8
                vs = []
                for cc in range(8):
                    c = c_base + cc
                    for db in range(dim // _L):
                        vs.append((c, db, rv[c, pl.ds(db * _L, _L)]))
                for c, db, v in vs:
                    cvec = jnp.zeros((_L,), jnp.int32) + c
                    plsc.store_scatter(rt, [dvecs[db], cvec], v)
                return carry

            lax.fori_loop(0, _BB // 8, col_body, 0)

        def fire_writes(g, b):
            cid = cid0 + g
            h = cid // nblk
            tc = cid - h * nblk
            for do in range(dt):
                pltpu.async_copy(rt_v.at[b, pl.ds(do * 8, 8), pl.ds(0, _BB)],
                                 out_hbm.at[h, do, tc], wsems[b])

        def drain_writes(b):
            pltpu.make_async_copy(tbl_hbm.at[pl.ds(0, _BB)], rows_v.at[0],
                                  wsems[b]).wait()

        for s in range(_NR):
            fire_idx(s, s)
        for s in range(_GD):
            wait_idx(s)
            fire_gather(s)

        def body(h_it, carry):
            g_base = _NR * h_it
            for s in range(_NR):
                g = g_base + s
                b = s % 2
                drain_gather(s)
                pl.when((h_it > 0) | (s >= 2))(lambda b=b: drain_writes(b))
                transpose(s, b)
                fire_writes(g, b)
                pl.when(g + _NR < per_w)(lambda g=g, s=s: fire_idx(g + _NR, s))
                u = (s + _GD) % _NR

                def launch(u=u):
                    wait_idx(u)
                    fire_gather(u)

                pl.when(g + _GD < per_w)(launch)
            return carry

        lax.fori_loop(0, per_w // _NR, body, 0)
        drain_writes(0)
        drain_writes(1)

    return emb(idx, table)


def kernel(input_ids, table):
    batch, hist = input_ids.shape
    vocab, dim = table.shape
    nblk = batch // _BB

    idx = input_ids.T.reshape(hist * nblk, _BB).astype(jnp.int32)
    out5 = _gather_call(idx, table, hist, nblk, dim)
    return out5.transpose(2, 4, 0, 1, 3).reshape(batch, hist, dim)
